# baseline (speedup 1.0000x reference)
import functools

import jax
import jax.numpy as jnp
from jax import lax
from jax.experimental import pallas as pl
from jax.experimental.pallas import tpu as pltpu
from jax.experimental.pallas import tpu_sc as plsc

VOCAB = 100000
EMBED = 64
CONTEXT = 5
BATCH = 1024
HIDDEN = 128

N_IDX = BATCH * 2 * CONTEXT
TILE_V = 2048

_info = plsc.get_sparse_core_info()
_NC, _NS = _info.num_cores, _info.num_subcores
_NW = _NC * _NS
_BPW = N_IDX // _NW
_CHUNK = 80


def _sc_gather(idx_hbm, table_hbm, out_hbm, idx_v, rows_v, sem):
    wid = lax.axis_index("s") * _NC + lax.axis_index("c")
    base = wid * _BPW
    pltpu.sync_copy(idx_hbm.at[pl.ds(base, _BPW)], idx_v)
    copies = []
    for c in range(_BPW // _CHUNK):
        copies.append(
            pltpu.async_copy(
                table_hbm.at[idx_v.at[pl.ds(c * _CHUNK, _CHUNK)]],
                rows_v.at[pl.ds(c * _CHUNK, _CHUNK)],
                sem,
            )
        )
    for cp in copies:
        cp.wait()
    pltpu.sync_copy(rows_v, out_hbm.at[pl.ds(base, _BPW)])


_gather_call = functools.partial(
    pl.kernel,
    mesh=plsc.VectorSubcoreMesh(core_axis_name="c", subcore_axis_name="s"),
    compiler_params=pltpu.CompilerParams(use_tc_tiling_on_sc=False),
    out_type=jax.ShapeDtypeStruct((N_IDX, EMBED), jnp.float32),
    scratch_types=[
        pltpu.VMEM((_BPW,), jnp.int32),
        pltpu.VMEM((_BPW, EMBED), jnp.float32),
        pltpu.SemaphoreType.DMA,
    ],
)(_sc_gather)


def _mlp_body(emb_ref, w1_ref, b1_ref, w2_ref, b2_ref, out_ref, h_ref):
    @pl.when(pl.program_id(0) == 0)
    def _():
        h = jnp.dot(emb_ref[...], w1_ref[...],
                    preferred_element_type=jnp.float32)
        h_ref[...] = jnp.maximum(h + b1_ref[...], 0.0)

    out_ref[...] = (
        jnp.dot(h_ref[...], w2_ref[...], preferred_element_type=jnp.float32)
        + b2_ref[...]
    )


def _mlp(emb, W1, b1, W2, b2):
    n_tiles = pl.cdiv(VOCAB, TILE_V)
    return pl.pallas_call(
        _mlp_body,
        grid=(n_tiles,),
        in_specs=[
            pl.BlockSpec((BATCH, 2 * CONTEXT * EMBED), lambda j: (0, 0)),
            pl.BlockSpec((2 * CONTEXT * EMBED, HIDDEN), lambda j: (0, 0)),
            pl.BlockSpec((1, HIDDEN), lambda j: (0, 0)),
            pl.BlockSpec((HIDDEN, TILE_V), lambda j: (0, j)),
            pl.BlockSpec((1, TILE_V), lambda j: (0, j)),
        ],
        out_specs=pl.BlockSpec((BATCH, TILE_V), lambda j: (0, j)),
        out_shape=jax.ShapeDtypeStruct((BATCH, VOCAB), jnp.float32),
        scratch_shapes=[pltpu.VMEM((BATCH, HIDDEN), jnp.float32)],
        compiler_params=pltpu.CompilerParams(
            dimension_semantics=("arbitrary",),
        ),
    )(emb, W1, b1, W2, b2)


def kernel(inputs, table, W1, b1, W2, b2):
    idx = inputs.reshape(-1).astype(jnp.int32)
    rows = _gather_call(idx, table)
    emb = rows.reshape(BATCH, 2 * CONTEXT * EMBED)
    return _mlp(emb, W1, b1.reshape(1, HIDDEN), W2, b2.reshape(1, VOCAB))
